# baseline (speedup 1.0000x reference)
import functools

import jax
import jax.numpy as jnp
from jax import lax
from jax.experimental import pallas as pl
from jax.experimental.pallas import tpu as pltpu
from jax.experimental.pallas import tpu_sc as plsc

BATCH = 16384

_INFO = plsc.get_sparse_core_info()
NC = _INFO.num_cores
NS = _INFO.num_subcores
L = _INFO.num_lanes
NW = NC * NS
BPW = BATCH // NW
CHUNK = 128
NCH = BPW // CHUNK
G = CHUNK // L

_mesh = plsc.VectorSubcoreMesh(core_axis_name="c", subcore_axis_name="s")


@functools.partial(
    pl.kernel,
    mesh=_mesh,
    out_type=jax.ShapeDtypeStruct((BATCH,), jnp.float32),
    scratch_types=[
        pltpu.VMEM((NCH, CHUNK), jnp.int32),
        pltpu.VMEM((NCH, CHUNK), jnp.int32),
        pltpu.VMEM((NCH, CHUNK), jnp.int32),
        pltpu.VMEM((NCH, CHUNK), jnp.int32),
        pltpu.VMEM((NCH, CHUNK), jnp.int32),
        pltpu.VMEM((NCH, CHUNK), jnp.int32),
        pltpu.VMEM((BPW,), jnp.float32),
        pltpu.VMEM((BPW,), jnp.float32),
        pltpu.VMEM((BPW,), jnp.float32),
        pltpu.VMEM((BPW,), jnp.float32),
        pltpu.VMEM((BPW,), jnp.float32),
        pltpu.VMEM((BPW,), jnp.float32),
        pltpu.SemaphoreType.DMA,
    ],
)
def _mirt_sc(stu_hbm, exer_hbm, theta_hbm, alpha_hbm, beta_hbm, out_hbm,
             idx_s, idx_e, ix_t0, ix_t1, ix_a0, ix_a1,
             t0_v, t1_v, a0_v, a1_v, be_v, out_v, sem):
    wid = lax.axis_index("s") * NC + lax.axis_index("c")
    base = wid * BPW

    for j in range(NCH):
        pltpu.sync_copy(stu_hbm.at[pl.ds(base + j * CHUNK, CHUNK)], idx_s.at[j])
        pltpu.sync_copy(exer_hbm.at[pl.ds(base + j * CHUNK, CHUNK)], idx_e.at[j])

    one = jnp.ones((L,), jnp.int32)
    for j in range(NCH):
        for g in range(G):
            sl = pl.ds(g * L, L)
            v = idx_s[j, sl]
            v2 = v + v
            ix_t0[j, sl] = v2
            ix_t1[j, sl] = v2 + one
            w = idx_e[j, sl]
            w2 = w + w
            ix_a0[j, sl] = w2
            ix_a1[j, sl] = w2 + one

    copies = []
    for j in range(NCH):
        sl = pl.ds(j * CHUNK, CHUNK)
        copies.append(pltpu.async_copy(theta_hbm.at[ix_t0.at[j]], t0_v.at[sl], sem))
        copies.append(pltpu.async_copy(theta_hbm.at[ix_t1.at[j]], t1_v.at[sl], sem))
        copies.append(pltpu.async_copy(alpha_hbm.at[ix_a0.at[j]], a0_v.at[sl], sem))
        copies.append(pltpu.async_copy(alpha_hbm.at[ix_a1.at[j]], a1_v.at[sl], sem))
        copies.append(pltpu.async_copy(beta_hbm.at[idx_e.at[j]], be_v.at[sl], sem))
    for c in copies:
        c.wait()

    for g in range(BPW // L):
        sl = pl.ds(g * L, L)
        t0 = t0_v[sl]
        t1 = t1_v[sl]
        a0 = a0_v[sl]
        a1 = a1_v[sl]
        b = be_v[sl]
        pred = a0 * (t0 - b) + a1 * (t1 - b)
        out_v[sl] = 1.0 / (1.0 + jnp.exp(-pred))

    pltpu.sync_copy(out_v, out_hbm.at[pl.ds(base, BPW)])


def kernel(stu_id, exer_id, theta_table, alpha_table, beta_table):
    return _mirt_sc(
        stu_id.astype(jnp.int32),
        exer_id.astype(jnp.int32),
        jnp.reshape(theta_table, (-1,)),
        jnp.reshape(alpha_table, (-1,)),
        jnp.reshape(beta_table, (-1,)),
    )
